# baseline (speedup 1.0000x reference)
import functools

import jax
import jax.numpy as jnp
from jax import lax
from jax.experimental import pallas as pl
from jax.experimental.pallas import tpu as pltpu
from jax.experimental.pallas import tpu_sc as plsc

B = 64
N = 512 * 512
NC = 2
NS = 16
NW = NC * NS
SAMPLES_PER_W = B // NW
CHUNK = 16384
CHUNKS_PER_SAMPLE = N // CHUNK
TOTAL_CHUNKS = SAMPLES_PER_W * CHUNKS_PER_SAMPLE
LANES = 16
UNROLL = 16
NEG = -1e30


@functools.partial(
    pl.kernel,
    out_type=jax.ShapeDtypeStruct((NW, SAMPLES_PER_W, LANES), jnp.float32),
    mesh=plsc.VectorSubcoreMesh(core_axis_name="c", subcore_axis_name="s"),
    scratch_types=[
        pltpu.VMEM((CHUNK,), jnp.float32),
        pltpu.VMEM((CHUNK,), jnp.float32),
        pltpu.VMEM((CHUNK,), jnp.int32),
        pltpu.VMEM((CHUNK,), jnp.int32),
        pltpu.VMEM((LANES,), jnp.int32),
        pltpu.VMEM((SAMPLES_PER_W, LANES), jnp.float32),
        pltpu.SemaphoreType.DMA,
        pltpu.SemaphoreType.DMA,
        pltpu.SemaphoreType.DMA,
        pltpu.SemaphoreType.DMA,
    ],
)
def _sc_bag_reduce(x_hbm, z_hbm, catsb_hbm, out_hbm,
                   xb0, xb1, zb0, zb1, cat_v, res_v,
                   sx0, sx1, sz0, sz1):
    cid = lax.axis_index("c")
    sid = lax.axis_index("s")
    wid = sid * NC + cid
    first_sample = wid * SAMPLES_PER_W

    xbufs = (xb0, xb1)
    zbufs = (zb0, zb1)
    sxs = (sx0, sx1)
    szs = (sz0, sz1)

    def start(k):
        smp = first_sample + (k // CHUNKS_PER_SAMPLE)
        off = (k % CHUNKS_PER_SAMPLE) * CHUNK
        hx = pltpu.async_copy(x_hbm.at[smp, pl.ds(off, CHUNK)],
                              xbufs[k % 2], sxs[k % 2])
        hz = pltpu.async_copy(z_hbm.at[smp, pl.ds(off, CHUNK)],
                              zbufs[k % 2], szs[k % 2])
        return hx, hz

    handles = start(0)
    vmax = jnp.full((LANES,), NEG, dtype=jnp.float32)
    cat_vec = None

    for k in range(TOTAL_CHUNKS):
        if k % CHUNKS_PER_SAMPLE == 0:
            smp = first_sample + (k // CHUNKS_PER_SAMPLE)
            pltpu.sync_copy(catsb_hbm.at[smp], cat_v)
            cat_vec = cat_v[...]
        nxt = start(k + 1) if k + 1 < TOTAL_CHUNKS else None
        handles[0].wait()
        handles[1].wait()
        xb = xbufs[k % 2]
        zb = zbufs[k % 2]

        def step(i, vm, xb=xb, zb=zb, cat_vec=cat_vec):
            base = i * (LANES * UNROLL)
            for u in range(UNROLL):
                z = zb[pl.ds(base + u * LANES, LANES)]
                x = xb[pl.ds(base + u * LANES, LANES)]
                vm = jnp.maximum(vm, jnp.where(z == cat_vec, x, NEG))
            return vm

        vmax = lax.fori_loop(0, CHUNK // (LANES * UNROLL), step, vmax)
        handles = nxt
        if (k + 1) % CHUNKS_PER_SAMPLE == 0:
            j = k // CHUNKS_PER_SAMPLE
            res_v[j, :] = vmax
            vmax = jnp.full((LANES,), NEG, dtype=jnp.float32)

    pltpu.sync_copy(res_v, out_hbm.at[wid])


def _loss_body(bagv_ref, lab_ref, out_ref):
    bag = jnp.max(bagv_ref[...], axis=1)
    x = jnp.where(bag > -1e29, bag, 0.0)
    y = lab_ref[...]
    per = jnp.maximum(x, 0.0) - x * y + jnp.log1p(jnp.exp(-jnp.abs(x)))
    out_ref[0, 0] = jnp.sum(per) / B


def kernel(pixel_logits, zone_patches, cats, labels):
    x = pixel_logits.reshape(B, N)
    z = zone_patches.reshape(B, N)
    cats_eff = jnp.where(cats > 0, cats, -1)
    cats_b = jnp.broadcast_to(cats_eff[:, None], (B, LANES))
    res = _sc_bag_reduce(x, z, cats_b)
    bagv = res.reshape(B, LANES)
    loss = pl.pallas_call(
        _loss_body,
        out_shape=jax.ShapeDtypeStruct((1, 1), jnp.float32),
        out_specs=pl.BlockSpec(memory_space=pltpu.SMEM),
    )(bagv, labels)
    return loss[0, 0]


# speedup vs baseline: 2.4065x; 2.4065x over previous
import functools

import jax
import jax.numpy as jnp
from jax import lax
from jax.experimental import pallas as pl
from jax.experimental.pallas import tpu as pltpu
from jax.experimental.pallas import tpu_sc as plsc

B = 64
H = 512
W = 512
NC = 2
NS = 16
NW = NC * NS
SAMPLES_PER_W = B // NW
ROWS = 32
CHUNKS_PER_SAMPLE = H // ROWS
CC = W // 16
LANES = 16
NEG = -1e30


@functools.partial(
    pl.kernel,
    out_type=jax.ShapeDtypeStruct((NW, SAMPLES_PER_W, LANES), jnp.float32),
    mesh=plsc.VectorSubcoreMesh(core_axis_name="c", subcore_axis_name="s"),
    scratch_types=[
        pltpu.VMEM((ROWS, W), jnp.float32),
        pltpu.VMEM((ROWS, W), jnp.float32),
        pltpu.VMEM((ROWS, W), jnp.int32),
        pltpu.VMEM((ROWS, W), jnp.int32),
        pltpu.VMEM((LANES,), jnp.int32),
        pltpu.VMEM((SAMPLES_PER_W, LANES), jnp.float32),
        pltpu.SemaphoreType.DMA,
        pltpu.SemaphoreType.DMA,
        pltpu.SemaphoreType.DMA,
        pltpu.SemaphoreType.DMA,
    ],
)
def _sc_bag_reduce(x_hbm, z_hbm, catsb_hbm, out_hbm,
                   xb0, xb1, zb0, zb1, cat_v, res_v,
                   sx0, sx1, sz0, sz1):
    cid = lax.axis_index("c")
    sid = lax.axis_index("s")
    wid = sid * NC + cid
    first_sample = wid * SAMPLES_PER_W

    xbufs = (xb0, xb1)
    zbufs = (zb0, zb1)
    sxs = (sx0, sx1)
    szs = (sz0, sz1)

    def start(smp, k, par):
        pltpu.async_copy(x_hbm.at[smp, pl.ds(k * ROWS, ROWS), :],
                         xbufs[par], sxs[par])
        pltpu.async_copy(z_hbm.at[smp, pl.ds(k * ROWS, ROWS), :],
                         zbufs[par], szs[par])

    def wait(smp, k, par):
        pltpu.make_async_copy(x_hbm.at[smp, pl.ds(k * ROWS, ROWS), :],
                              xbufs[par], sxs[par]).wait()
        pltpu.make_async_copy(z_hbm.at[smp, pl.ds(k * ROWS, ROWS), :],
                              zbufs[par], szs[par]).wait()

    for j in range(SAMPLES_PER_W):
        smp = first_sample + j
        pltpu.sync_copy(catsb_hbm.at[smp], cat_v)
        cat_vec = cat_v[...]

        start(smp, 0, 0)
        start(smp, 1, 1)

        def chunk_pair(g, vm, smp=smp, cat_vec=cat_vec):
            for par in range(2):
                k = 2 * g + par
                wait(smp, k, par)
                xb = xbufs[par]
                zb = zbufs[par]

                def row_body(r, vmr, xb=xb, zb=zb, cat_vec=cat_vec):
                    for c in range(CC):
                        z = zb[r, pl.ds(c * LANES, LANES)]
                        x = xb[r, pl.ds(c * LANES, LANES)]
                        vmr = jnp.maximum(vmr,
                                          jnp.where(z == cat_vec, x, NEG))
                    return vmr

                vm = lax.fori_loop(0, ROWS, row_body, vm)

                @pl.when(k + 2 < CHUNKS_PER_SAMPLE)
                def _(smp=smp, k=k, par=par):
                    start(smp, k + 2, par)
            return vm

        vmax = lax.fori_loop(0, CHUNKS_PER_SAMPLE // 2, chunk_pair,
                             jnp.full((LANES,), NEG, dtype=jnp.float32))
        res_v[j, :] = vmax

    pltpu.sync_copy(res_v, out_hbm.at[wid])


def _loss_body(bagv_ref, lab_ref, out_ref):
    bag = jnp.max(bagv_ref[...], axis=1)
    x = jnp.where(bag > -1e29, bag, 0.0)
    y = lab_ref[...]
    per = jnp.maximum(x, 0.0) - x * y + jnp.log1p(jnp.exp(-jnp.abs(x)))
    out_ref[0, 0] = jnp.sum(per) / B


def kernel(pixel_logits, zone_patches, cats, labels):
    x = pixel_logits.reshape(B, H, W)
    z = zone_patches
    cats_eff = jnp.where(cats > 0, cats, -1)
    cats_b = jnp.broadcast_to(cats_eff[:, None], (B, LANES))
    res = _sc_bag_reduce(x, z, cats_b)
    bagv = res.reshape(B, LANES)
    loss = pl.pallas_call(
        _loss_body,
        out_shape=jax.ShapeDtypeStruct((1, 1), jnp.float32),
        out_specs=pl.BlockSpec(memory_space=pltpu.SMEM),
    )(bagv, labels)
    return loss[0, 0]
